# baseline (speedup 1.0000x reference)
import functools

import jax
import jax.numpy as jnp
from jax import lax
from jax.experimental import pallas as pl
from jax.experimental.pallas import tpu as pltpu
from jax.experimental.pallas import tpu_sc as plsc

LR = 0.01
WD = 1e-4
NUM_USERS = 100000
NUM_ITEMS = 1000000
D = 64
B = 16384

NC = 2
NS = 16
CHUNK = 128
PER_SUB = B // NS
NCHUNK = PER_SUB // CHUNK

MBLK = 256
DBLK = 4000
N_UB = NUM_USERS // DBLK
N_IB = NUM_ITEMS // DBLK

_mesh = plsc.VectorSubcoreMesh(core_axis_name="c", subcore_axis_name="s")
_sc_params = pltpu.CompilerParams(use_tc_tiling_on_sc=False)


@functools.partial(
    pl.kernel,
    out_type=jax.ShapeDtypeStruct((2 * B, D), jnp.float32),
    mesh=_mesh,
    scratch_types=[
        pltpu.VMEM((1, CHUNK), jnp.int32),
        pltpu.VMEM((CHUNK, D), jnp.float32),
    ],
    compiler_params=_sc_params,
)
def _sc_gather(uemb, iemb, idxg, gout, idx_v, rows_v):
    t = lax.axis_index("c")
    sub = lax.axis_index("s")

    @pl.loop(0, NCHUNK)
    def _(k):
        b0 = sub * PER_SUB + k * CHUNK
        pltpu.sync_copy(idxg.at[t, pl.ds(b0, CHUNK)], idx_v.at[0])

        @pl.when(t == 0)
        def _():
            pltpu.sync_copy(uemb.at[idx_v.at[0]], rows_v)

        @pl.when(t == 1)
        def _():
            pltpu.sync_copy(iemb.at[idx_v.at[0]], rows_v)

        pltpu.sync_copy(rows_v, gout.at[pl.ds(t * B + b0, CHUNK)])


def _match_body(idx_col_ref, idx_row_ref, gradp_ref, grows_ref, out_ref):
    a = idx_col_ref[...]
    b = idx_row_ref[...]
    m = (a == b).astype(jnp.bfloat16)
    s = lax.dot_general(
        m, gradp_ref[...],
        dimension_numbers=(((1,), (0,)), ((), ())),
        preferred_element_type=jnp.float32,
    )
    cnt = s[:, D:D + 1]
    out_ref[...] = grows_ref[...] * (1.0 - WD) - (LR / cnt) * s[:, :D]


def _tc_match(idx, gradp, grows):
    return pl.pallas_call(
        _match_body,
        grid=(B // MBLK,),
        in_specs=[
            pl.BlockSpec((MBLK, 1), lambda i: (i, 0)),
            pl.BlockSpec((1, B), lambda i: (0, 0)),
            pl.BlockSpec((B, 2 * D), lambda i: (0, 0)),
            pl.BlockSpec((MBLK, D), lambda i: (i, 0)),
        ],
        out_specs=pl.BlockSpec((MBLK, D), lambda i: (i, 0)),
        out_shape=jax.ShapeDtypeStruct((B, D), jnp.float32),
    )(idx.reshape(B, 1), idx.reshape(1, B), gradp, grows)


def _dense_body(u_ref, i_ref, o_ref):
    step = pl.program_id(0)

    @pl.when(step < N_UB)
    def _():
        o_ref[...] = u_ref[...] * (1.0 - WD)

    @pl.when(step >= N_UB)
    def _():
        o_ref[...] = i_ref[...] * (1.0 - WD)


def _tc_dense(user_emb, item_emb):
    return pl.pallas_call(
        _dense_body,
        grid=(N_UB + N_IB,),
        in_specs=[
            pl.BlockSpec((DBLK, D), lambda i: (jnp.minimum(i, N_UB - 1), 0)),
            pl.BlockSpec((DBLK, D), lambda i: (jnp.maximum(i - N_UB, 0), 0)),
        ],
        out_specs=pl.BlockSpec((DBLK, D), lambda i: (i, 0)),
        out_shape=jax.ShapeDtypeStruct((NUM_USERS + NUM_ITEMS, D), jnp.float32),
    )(user_emb, item_emb)


@functools.partial(
    pl.kernel,
    out_type=(),
    mesh=_mesh,
    scratch_types=[
        pltpu.VMEM((1, CHUNK), jnp.int32),
        pltpu.VMEM((CHUNK, D), jnp.float32),
    ],
    compiler_params=_sc_params,
)
def _sc_scatter(idxs, val_u, val_i, out_hbm, idx_v, rows_v):
    t = lax.axis_index("c")
    sub = lax.axis_index("s")

    @pl.loop(0, NCHUNK)
    def _(k):
        b0 = sub * PER_SUB + k * CHUNK
        pltpu.sync_copy(idxs.at[t, pl.ds(b0, CHUNK)], idx_v.at[0])

        @pl.when(t == 0)
        def _():
            pltpu.sync_copy(val_u.at[pl.ds(b0, CHUNK)], rows_v)

        @pl.when(t == 1)
        def _():
            pltpu.sync_copy(val_i.at[pl.ds(b0, CHUNK)], rows_v)

        pltpu.sync_copy(rows_v, out_hbm.at[idx_v.at[0]])


def _pad_grad(grad):
    ones = jnp.ones((B, 1), jnp.bfloat16)
    zeros = jnp.zeros((B, D - 1), jnp.bfloat16)
    return jnp.concatenate([grad.astype(jnp.bfloat16), ones, zeros], axis=1)


def kernel(user_emb, item_emb, user_grad, item_grad, returned_users, returned_items):
    idxg = jnp.stack([returned_users, returned_items])
    idxs = jnp.stack([returned_users, returned_items + NUM_USERS])

    grows = _sc_gather(user_emb, item_emb, idxg)
    val_u = _tc_match(returned_users, _pad_grad(user_grad), grows[:B])
    val_i = _tc_match(returned_items, _pad_grad(item_grad), grows[B:])

    out = _tc_dense(user_emb, item_emb)
    out_ref = jax.new_ref(out)
    _sc_scatter(idxs, val_u, val_i, out_ref)
    return out_ref[...]


# speedup vs baseline: 1.7337x; 1.7337x over previous
import functools

import jax
import jax.numpy as jnp
from jax import lax
from jax.experimental import pallas as pl
from jax.experimental.pallas import tpu as pltpu
from jax.experimental.pallas import tpu_sc as plsc

LR = 0.01
WD = 1e-4
NUM_USERS = 100000
NUM_ITEMS = 1000000
D = 64
B = 16384

NC = 2
NS = 16
CHUNK = 128
PER_SUB = B // NS
NCHUNK = PER_SUB // CHUNK

MBLK = 256
DBLK = 4000
N_UB = NUM_USERS // DBLK
N_IB = NUM_ITEMS // DBLK

_mesh = plsc.VectorSubcoreMesh(core_axis_name="c", subcore_axis_name="s")
GRP = 16


def _rowwise_dma(table, idx_v, rows_v, sem, to_table):
    for g in range(CHUNK // GRP):
        vec = idx_v[0, pl.ds(g * GRP, GRP)]
        for j in range(GRP):
            r = vec[j]
            if to_table:
                pltpu.async_copy(rows_v.at[g * GRP + j], table.at[r], sem)
            else:
                pltpu.async_copy(table.at[r], rows_v.at[g * GRP + j], sem)
        pltpu.make_async_copy(
            table.at[pl.ds(0, GRP)], rows_v.at[pl.ds(g * GRP, GRP)], sem
        ).wait()


@functools.partial(
    pl.kernel,
    out_type=jax.ShapeDtypeStruct((2 * B, D), jnp.float32),
    mesh=_mesh,
    scratch_types=[
        pltpu.VMEM((1, CHUNK), jnp.int32),
        pltpu.VMEM((CHUNK, D), jnp.float32),
        pltpu.SemaphoreType.DMA,
    ],
)
def _sc_gather(uemb, iemb, idxg, gout, idx_v, rows_v, sem):
    t = lax.axis_index("c")
    sub = lax.axis_index("s")

    @pl.loop(0, NCHUNK)
    def _(k):
        b0 = sub * PER_SUB + k * CHUNK
        pltpu.sync_copy(idxg.at[t, pl.ds(b0, CHUNK)], idx_v.at[0])

        @pl.when(t == 0)
        def _():
            _rowwise_dma(uemb, idx_v, rows_v, sem, to_table=False)

        @pl.when(t == 1)
        def _():
            _rowwise_dma(iemb, idx_v, rows_v, sem, to_table=False)

        pltpu.sync_copy(rows_v, gout.at[pl.ds(t * B + b0, CHUNK)])


def _match_body(idx_col_ref, idx_row_ref, gradp_ref, grows_ref, out_ref):
    a = idx_col_ref[...]
    b = idx_row_ref[...]
    m = (a == b).astype(jnp.bfloat16)
    s = lax.dot_general(
        m, gradp_ref[...],
        dimension_numbers=(((1,), (0,)), ((), ())),
        preferred_element_type=jnp.float32,
    )
    cnt = s[:, D:D + 1]
    out_ref[...] = grows_ref[...] * (1.0 - WD) - (LR / cnt) * s[:, :D]


def _tc_match(idx, gradp, grows):
    return pl.pallas_call(
        _match_body,
        grid=(B // MBLK,),
        in_specs=[
            pl.BlockSpec((MBLK, 1), lambda i: (i, 0)),
            pl.BlockSpec((1, B), lambda i: (0, 0)),
            pl.BlockSpec((B, 2 * D), lambda i: (0, 0)),
            pl.BlockSpec((MBLK, D), lambda i: (i, 0)),
        ],
        out_specs=pl.BlockSpec((MBLK, D), lambda i: (i, 0)),
        out_shape=jax.ShapeDtypeStruct((B, D), jnp.float32),
    )(idx.reshape(B, 1), idx.reshape(1, B), gradp, grows)


def _dense_body(u_ref, i_ref, o_ref):
    step = pl.program_id(0)

    @pl.when(step < N_UB)
    def _():
        o_ref[...] = u_ref[...] * (1.0 - WD)

    @pl.when(step >= N_UB)
    def _():
        o_ref[...] = i_ref[...] * (1.0 - WD)


def _tc_dense(user_emb, item_emb):
    return pl.pallas_call(
        _dense_body,
        grid=(N_UB + N_IB,),
        in_specs=[
            pl.BlockSpec((DBLK, D), lambda i: (jnp.minimum(i, N_UB - 1), 0)),
            pl.BlockSpec((DBLK, D), lambda i: (jnp.maximum(i - N_UB, 0), 0)),
        ],
        out_specs=pl.BlockSpec((DBLK, D), lambda i: (i, 0)),
        out_shape=jax.ShapeDtypeStruct((NUM_USERS + NUM_ITEMS, D), jnp.float32),
    )(user_emb, item_emb)


@functools.partial(
    pl.kernel,
    out_type=(),
    mesh=_mesh,
    scratch_types=[
        pltpu.VMEM((1, CHUNK), jnp.int32),
        pltpu.VMEM((CHUNK, D), jnp.float32),
        pltpu.SemaphoreType.DMA,
    ],
)
def _sc_scatter(idxs, val_u, val_i, out_hbm, idx_v, rows_v, sem):
    t = lax.axis_index("c")
    sub = lax.axis_index("s")

    @pl.loop(0, NCHUNK)
    def _(k):
        b0 = sub * PER_SUB + k * CHUNK
        pltpu.sync_copy(idxs.at[t, pl.ds(b0, CHUNK)], idx_v.at[0])

        @pl.when(t == 0)
        def _():
            pltpu.sync_copy(val_u.at[pl.ds(b0, CHUNK)], rows_v)

        @pl.when(t == 1)
        def _():
            pltpu.sync_copy(val_i.at[pl.ds(b0, CHUNK)], rows_v)

        _rowwise_dma(out_hbm, idx_v, rows_v, sem, to_table=True)


def _pad_grad(grad):
    ones = jnp.ones((B, 1), jnp.bfloat16)
    zeros = jnp.zeros((B, D - 1), jnp.bfloat16)
    return jnp.concatenate([grad.astype(jnp.bfloat16), ones, zeros], axis=1)


def kernel(user_emb, item_emb, user_grad, item_grad, returned_users, returned_items):
    idxg = jnp.stack([returned_users, returned_items])
    idxs = jnp.stack([returned_users, returned_items + NUM_USERS])

    grows = _sc_gather(user_emb, item_emb, idxg)
    val_u = _tc_match(returned_users, _pad_grad(user_grad), grows[:B])
    val_i = _tc_match(returned_items, _pad_grad(item_grad), grows[B:])

    out = _tc_dense(user_emb, item_emb)
    out_ref = jax.new_ref(out)
    _sc_scatter(idxs, val_u, val_i, out_ref)
    return out_ref[...]
